# baseline (speedup 1.0000x reference)
import functools

import jax
import jax.numpy as jnp
from jax.experimental import pallas as pl
from jax.experimental.pallas import tpu as pltpu

N_EXPERTS = 8
TOP_K = 2
TOKEN_BLOCK = 1024


def _router_kernel(x_ref, w_ref, probs_ref, idx_ref, vals_ref):
    x = x_ref[...]
    w = w_ref[...]
    logits = jax.lax.dot_general(
        x, w, (((1,), (1,)), ((), ())), preferred_element_type=jnp.float32
    )
    m = jnp.max(logits, axis=-1, keepdims=True)
    e = jnp.exp(logits - m)
    s = jnp.sum(e, axis=-1, keepdims=True)
    probs = e / s
    probs_ref[...] = probs

    col = jax.lax.broadcasted_iota(jnp.int32, probs.shape, 1)
    v1 = jnp.max(probs, axis=-1, keepdims=True)
    i1 = jnp.min(jnp.where(probs == v1, col, N_EXPERTS), axis=-1, keepdims=True)
    masked = jnp.where(col == i1, -jnp.inf, probs)
    v2 = jnp.max(masked, axis=-1, keepdims=True)
    i2 = jnp.min(jnp.where(masked == v2, col, N_EXPERTS), axis=-1, keepdims=True)
    idx_ref[...] = jnp.concatenate([i1, i2], axis=-1)
    vals_ref[...] = jnp.concatenate([v1, v2], axis=-1)


@functools.partial(jax.jit, static_argnames=())
def kernel(x, w):
    n_tokens, d_model = x.shape
    grid = (n_tokens // TOKEN_BLOCK,)
    out_shapes = (
        jax.ShapeDtypeStruct((n_tokens, N_EXPERTS), jnp.float32),
        jax.ShapeDtypeStruct((n_tokens, TOP_K), jnp.int32),
        jax.ShapeDtypeStruct((n_tokens, TOP_K), jnp.float32),
    )
    probs, idx, vals = pl.pallas_call(
        _router_kernel,
        grid=grid,
        in_specs=[
            pl.BlockSpec((TOKEN_BLOCK, d_model), lambda i: (i, 0)),
            pl.BlockSpec((N_EXPERTS, d_model), lambda i: (0, 0)),
        ],
        out_specs=(
            pl.BlockSpec((TOKEN_BLOCK, N_EXPERTS), lambda i: (i, 0)),
            pl.BlockSpec((TOKEN_BLOCK, TOP_K), lambda i: (i, 0)),
            pl.BlockSpec((TOKEN_BLOCK, TOP_K), lambda i: (i, 0)),
        ),
        out_shape=out_shapes,
        compiler_params=pltpu.CompilerParams(
            dimension_semantics=("parallel",),
        ),
    )(x, w)
    return (probs, idx, vals)


# speedup vs baseline: 1.1266x; 1.1266x over previous
import functools

import jax
import jax.numpy as jnp
from jax.experimental import pallas as pl
from jax.experimental.pallas import tpu as pltpu

N_EXPERTS = 8
TOP_K = 2
TOKEN_BLOCK = 1024


def _router_kernel(x_ref, w_ref, probs_ref, idx_ref, vals_ref):
    x = x_ref[...]
    w = w_ref[...]
    lg = jax.lax.dot_general(
        w, x, (((1,), (1,)), ((), ())), preferred_element_type=jnp.float32
    )
    m = jnp.max(lg, axis=0, keepdims=True)
    e = jnp.exp(lg - m)
    s = jnp.sum(e, axis=0, keepdims=True)
    p = e / s
    probs_ref[...] = p.T

    row = jax.lax.broadcasted_iota(jnp.int32, p.shape, 0)
    v1 = jnp.max(p, axis=0, keepdims=True)
    i1 = jnp.min(jnp.where(p == v1, row, N_EXPERTS), axis=0, keepdims=True)
    masked = jnp.where(row == i1, -jnp.inf, p)
    v2 = jnp.max(masked, axis=0, keepdims=True)
    i2 = jnp.min(jnp.where(masked == v2, row, N_EXPERTS), axis=0, keepdims=True)
    idx_ref[...] = jnp.concatenate([i1, i2], axis=0).T
    vals_ref[...] = jnp.concatenate([v1, v2], axis=0).T


@functools.partial(jax.jit, static_argnames=())
def kernel(x, w):
    n_tokens, d_model = x.shape
    grid = (n_tokens // TOKEN_BLOCK,)
    out_shapes = (
        jax.ShapeDtypeStruct((n_tokens, N_EXPERTS), jnp.float32),
        jax.ShapeDtypeStruct((n_tokens, TOP_K), jnp.int32),
        jax.ShapeDtypeStruct((n_tokens, TOP_K), jnp.float32),
    )
    probs, idx, vals = pl.pallas_call(
        _router_kernel,
        grid=grid,
        in_specs=[
            pl.BlockSpec((TOKEN_BLOCK, d_model), lambda i: (i, 0)),
            pl.BlockSpec((N_EXPERTS, d_model), lambda i: (0, 0)),
        ],
        out_specs=(
            pl.BlockSpec((TOKEN_BLOCK, N_EXPERTS), lambda i: (i, 0)),
            pl.BlockSpec((TOKEN_BLOCK, TOP_K), lambda i: (i, 0)),
            pl.BlockSpec((TOKEN_BLOCK, TOP_K), lambda i: (i, 0)),
        ),
        out_shape=out_shapes,
        compiler_params=pltpu.CompilerParams(
            dimension_semantics=("parallel",),
        ),
    )(x, w)
    return (probs, idx, vals)


# speedup vs baseline: 2.2161x; 1.9671x over previous
import functools

import jax
import jax.numpy as jnp
from jax.experimental import pallas as pl
from jax.experimental.pallas import tpu as pltpu

N_EXPERTS = 8
TOP_K = 2
TOKEN_BLOCK = 1024


def _router_kernel(x_ref, w_ref, probs_ref, idx_ref, vals_ref):
    x = x_ref[...]
    w = w_ref[...]
    lg = jax.lax.dot_general(
        w, x, (((1,), (1,)), ((), ())), preferred_element_type=jnp.float32
    )
    m = jnp.max(lg, axis=0, keepdims=True)
    e = jnp.exp(lg - m)
    s = jnp.sum(e, axis=0, keepdims=True)
    p = e / s
    probs_ref[...] = p

    row = jax.lax.broadcasted_iota(jnp.int32, p.shape, 0)
    v1 = jnp.max(p, axis=0, keepdims=True)
    i1 = jnp.min(jnp.where(p == v1, row, N_EXPERTS), axis=0, keepdims=True)
    masked = jnp.where(row == i1, -jnp.inf, p)
    v2 = jnp.max(masked, axis=0, keepdims=True)
    i2 = jnp.min(jnp.where(masked == v2, row, N_EXPERTS), axis=0, keepdims=True)
    idx_ref[...] = jnp.concatenate([i1, i2], axis=0)
    vals_ref[...] = jnp.concatenate([v1, v2], axis=0)


@functools.partial(jax.jit, static_argnames=())
def kernel(x, w):
    n_tokens, d_model = x.shape
    grid = (n_tokens // TOKEN_BLOCK,)
    out_shapes = (
        jax.ShapeDtypeStruct((N_EXPERTS, n_tokens), jnp.float32),
        jax.ShapeDtypeStruct((TOP_K, n_tokens), jnp.int32),
        jax.ShapeDtypeStruct((TOP_K, n_tokens), jnp.float32),
    )
    probs_t, idx_t, vals_t = pl.pallas_call(
        _router_kernel,
        grid=grid,
        in_specs=[
            pl.BlockSpec((TOKEN_BLOCK, d_model), lambda i: (i, 0)),
            pl.BlockSpec((N_EXPERTS, d_model), lambda i: (0, 0)),
        ],
        out_specs=(
            pl.BlockSpec((N_EXPERTS, TOKEN_BLOCK), lambda i: (0, i)),
            pl.BlockSpec((TOP_K, TOKEN_BLOCK), lambda i: (0, i)),
            pl.BlockSpec((TOP_K, TOKEN_BLOCK), lambda i: (0, i)),
        ),
        out_shape=out_shapes,
        compiler_params=pltpu.CompilerParams(
            dimension_semantics=("parallel",),
        ),
    )(x, w)
    return (probs_t.T, idx_t.T, vals_t.T)


# speedup vs baseline: 2.7995x; 1.2632x over previous
import functools

import jax
import jax.numpy as jnp
from jax.experimental import pallas as pl
from jax.experimental.pallas import tpu as pltpu

N_EXPERTS = 8
TOP_K = 2
TOKEN_BLOCK = 2048


def _router_kernel(x_ref, w_ref, probs_ref, idx_ref, vals_ref):
    x = x_ref[...]
    w = w_ref[...]
    lg = jax.lax.dot_general(
        w, x, (((1,), (1,)), ((), ())), preferred_element_type=jnp.float32
    )
    m = jnp.max(lg, axis=0, keepdims=True)
    e = jnp.exp(lg - m)
    s = jnp.sum(e, axis=0, keepdims=True)
    p = e / s
    probs_ref[...] = p

    row = jax.lax.broadcasted_iota(jnp.int32, p.shape, 0)
    v1 = jnp.max(p, axis=0, keepdims=True)
    i1 = jnp.min(jnp.where(p == v1, row, N_EXPERTS), axis=0, keepdims=True)
    masked = jnp.where(row == i1, -jnp.inf, p)
    v2 = jnp.max(masked, axis=0, keepdims=True)
    i2 = jnp.min(jnp.where(masked == v2, row, N_EXPERTS), axis=0, keepdims=True)
    idx_ref[...] = jnp.concatenate([i1, i2], axis=0)
    vals_ref[...] = jnp.concatenate([v1, v2], axis=0)


@functools.partial(jax.jit, static_argnames=())
def kernel(x, w):
    n_tokens, d_model = x.shape
    grid = (n_tokens // TOKEN_BLOCK,)
    out_shapes = (
        jax.ShapeDtypeStruct((N_EXPERTS, n_tokens), jnp.float32),
        jax.ShapeDtypeStruct((TOP_K, n_tokens), jnp.int32),
        jax.ShapeDtypeStruct((TOP_K, n_tokens), jnp.float32),
    )
    probs_t, idx_t, vals_t = pl.pallas_call(
        _router_kernel,
        grid=grid,
        in_specs=[
            pl.BlockSpec((TOKEN_BLOCK, d_model), lambda i: (i, 0)),
            pl.BlockSpec((N_EXPERTS, d_model), lambda i: (0, 0)),
        ],
        out_specs=(
            pl.BlockSpec((N_EXPERTS, TOKEN_BLOCK), lambda i: (0, i)),
            pl.BlockSpec((TOP_K, TOKEN_BLOCK), lambda i: (0, i)),
            pl.BlockSpec((TOP_K, TOKEN_BLOCK), lambda i: (0, i)),
        ),
        out_shape=out_shapes,
        compiler_params=pltpu.CompilerParams(
            dimension_semantics=("parallel",),
        ),
    )(x, w)
    return (probs_t.T, idx_t.T, vals_t.T)


# speedup vs baseline: 2.8853x; 1.0306x over previous
import functools

import jax
import jax.numpy as jnp
from jax.experimental import pallas as pl
from jax.experimental.pallas import tpu as pltpu

N_EXPERTS = 8
TOP_K = 2
TOKEN_BLOCK = 4096


def _router_kernel(x_ref, w_ref, probs_ref, idx_ref, vals_ref):
    x = x_ref[...]
    w = w_ref[...]
    lg = jax.lax.dot_general(
        w, x, (((1,), (1,)), ((), ())), preferred_element_type=jnp.float32
    )
    m = jnp.max(lg, axis=0, keepdims=True)
    e = jnp.exp(lg - m)
    s = jnp.sum(e, axis=0, keepdims=True)
    p = e / s
    probs_ref[...] = p

    row = jax.lax.broadcasted_iota(jnp.int32, p.shape, 0)
    v1 = jnp.max(p, axis=0, keepdims=True)
    i1 = jnp.min(jnp.where(p == v1, row, N_EXPERTS), axis=0, keepdims=True)
    masked = jnp.where(row == i1, -jnp.inf, p)
    v2 = jnp.max(masked, axis=0, keepdims=True)
    i2 = jnp.min(jnp.where(masked == v2, row, N_EXPERTS), axis=0, keepdims=True)
    idx_ref[...] = jnp.concatenate([i1, i2], axis=0)
    vals_ref[...] = jnp.concatenate([v1, v2], axis=0)


@functools.partial(jax.jit, static_argnames=())
def kernel(x, w):
    n_tokens, d_model = x.shape
    grid = (n_tokens // TOKEN_BLOCK,)
    out_shapes = (
        jax.ShapeDtypeStruct((N_EXPERTS, n_tokens), jnp.float32),
        jax.ShapeDtypeStruct((TOP_K, n_tokens), jnp.int32),
        jax.ShapeDtypeStruct((TOP_K, n_tokens), jnp.float32),
    )
    probs_t, idx_t, vals_t = pl.pallas_call(
        _router_kernel,
        grid=grid,
        in_specs=[
            pl.BlockSpec((TOKEN_BLOCK, d_model), lambda i: (i, 0)),
            pl.BlockSpec((N_EXPERTS, d_model), lambda i: (0, 0)),
        ],
        out_specs=(
            pl.BlockSpec((N_EXPERTS, TOKEN_BLOCK), lambda i: (0, i)),
            pl.BlockSpec((TOP_K, TOKEN_BLOCK), lambda i: (0, i)),
            pl.BlockSpec((TOP_K, TOKEN_BLOCK), lambda i: (0, i)),
        ),
        out_shape=out_shapes,
        compiler_params=pltpu.CompilerParams(
            dimension_semantics=("parallel",),
        ),
    )(x, w)
    return (probs_t.T, idx_t.T, vals_t.T)
